# Initial kernel scaffold; baseline (speedup 1.0000x reference)
import jax
import jax.numpy as jnp
from jax.experimental import pallas as pl


def kernel(x, token_emb):
    raise NotImplementedError("write your pallas kernel here")


# speedup vs baseline: 4.9386x; 4.9386x over previous
import math
import functools

import jax
import jax.numpy as jnp
import numpy as np
from jax import lax
from jax.experimental import pallas as pl
from jax.experimental.pallas import tpu as pltpu
from jax.experimental.pallas import tpu_sc as plsc

VOCAB = 100000
D_MODEL = 128
SEQ = 200
BATCH = 4096

NUM_CORES = 2
NUM_SUBCORES = 16
NUM_WORKERS = NUM_CORES * NUM_SUBCORES

TOKENS = BATCH * SEQ
TOK_PER_W = TOKENS // NUM_WORKERS
SEQ_PER_W = TOK_PER_W // SEQ


def _positional_encoding():
    position = np.arange(0, SEQ, dtype=np.float64)[:, None]
    div_term = np.exp(
        np.arange(0, D_MODEL, 2, dtype=np.float64) * -(math.log(10000.0) / D_MODEL)
    )
    pe = np.zeros((SEQ, D_MODEL), dtype=np.float32)
    pe[:, 0::2] = np.sin(position * div_term).astype(np.float32)
    pe[:, 1::2] = np.cos(position * div_term).astype(np.float32)
    return pe


@functools.cache
def _build_emb_kernel():
    mesh = plsc.VectorSubcoreMesh(
        core_axis_name="c",
        subcore_axis_name="s",
        num_cores=NUM_CORES,
        num_subcores=NUM_SUBCORES,
    )
    return functools.partial(
        pl.kernel,
        out_type=jax.ShapeDtypeStruct((TOKENS, D_MODEL), jnp.float32),
        mesh=mesh,
        scratch_types=[
            pltpu.VMEM_SHARED((SEQ, D_MODEL), jnp.float32),
            pltpu.VMEM((SEQ,), jnp.int32),
            pltpu.VMEM((SEQ, D_MODEL), jnp.float32),
            pltpu.SemaphoreType.DMA,
        ],
    )(_emb_body)


def _emb_body(x_hbm, emb_hbm, pe_hbm, out_hbm, pe_sh, idx_v, rows_v, sem):
    cid = lax.axis_index("c")
    sid = lax.axis_index("s")
    wid = sid * NUM_CORES + cid

    @pl.when(sid == 0)
    def _():
        pltpu.sync_copy(pe_hbm, pe_sh)

    plsc.subcore_barrier()

    @pl.loop(0, SEQ_PER_W)
    def _chunk(c):
        base = wid * TOK_PER_W + c * SEQ
        pltpu.sync_copy(x_hbm.at[pl.ds(base, SEQ)], idx_v)
        pltpu.sync_copy(pe_sh, rows_v)
        d0 = pltpu.async_copy(
            emb_hbm.at[idx_v.at[pl.ds(0, 128)]],
            rows_v.at[pl.ds(0, 128)],
            sem,
            add=True,
        )
        d1 = pltpu.async_copy(
            emb_hbm.at[idx_v.at[pl.ds(128, SEQ - 128)]],
            rows_v.at[pl.ds(128, SEQ - 128)],
            sem,
            add=True,
        )
        d0.wait()
        d1.wait()
        pltpu.sync_copy(rows_v, out_hbm.at[pl.ds(base, SEQ)])


_PE = _positional_encoding()


def kernel(x, token_emb):
    out = _build_emb_kernel()(x.reshape(-1), token_emb, jnp.asarray(_PE))
    return out.reshape(BATCH, SEQ, D_MODEL)


# speedup vs baseline: 6.7743x; 1.3717x over previous
import math
import functools

import jax
import jax.numpy as jnp
import numpy as np
from jax import lax
from jax.experimental import pallas as pl
from jax.experimental.pallas import tpu as pltpu
from jax.experimental.pallas import tpu_sc as plsc

VOCAB = 100000
D_MODEL = 128
SEQ = 200
BATCH = 4096

NUM_CORES = 2
NUM_SUBCORES = 16
NUM_WORKERS = NUM_CORES * NUM_SUBCORES

TOKENS = BATCH * SEQ
TOK_PER_W = TOKENS // NUM_WORKERS
SEQ_PER_W = TOK_PER_W // SEQ


def _positional_encoding():
    position = np.arange(0, SEQ, dtype=np.float64)[:, None]
    div_term = np.exp(
        np.arange(0, D_MODEL, 2, dtype=np.float64) * -(math.log(10000.0) / D_MODEL)
    )
    pe = np.zeros((SEQ, D_MODEL), dtype=np.float32)
    pe[:, 0::2] = np.sin(position * div_term).astype(np.float32)
    pe[:, 1::2] = np.cos(position * div_term).astype(np.float32)
    return pe


@functools.cache
def _build_emb_kernel():
    mesh = plsc.VectorSubcoreMesh(
        core_axis_name="c",
        subcore_axis_name="s",
        num_cores=NUM_CORES,
        num_subcores=NUM_SUBCORES,
    )
    return functools.partial(
        pl.kernel,
        out_type=jax.ShapeDtypeStruct((TOKENS, D_MODEL), jnp.float32),
        mesh=mesh,
        scratch_types=[
            pltpu.VMEM_SHARED((SEQ, D_MODEL), jnp.float32),
            pltpu.VMEM((SEQ,), jnp.int32),
            pltpu.VMEM((SEQ,), jnp.int32),
            pltpu.VMEM((SEQ, D_MODEL), jnp.float32),
            pltpu.VMEM((SEQ, D_MODEL), jnp.float32),
            pltpu.SemaphoreType.DMA,
            pltpu.SemaphoreType.DMA,
            pltpu.SemaphoreType.DMA,
            pltpu.SemaphoreType.DMA,
            pltpu.SemaphoreType.DMA,
            pltpu.SemaphoreType.DMA,
            pltpu.SemaphoreType.DMA,
            pltpu.SemaphoreType.DMA,
        ],
    )(_emb_body)


def _emb_body(
    x_hbm, emb_hbm, pe_hbm, out_hbm, pe_sh,
    idx0, idx1, rows0, rows1,
    sem_pre0, sem_pre1, sem_idx0, sem_idx1, sem_g0, sem_g1, sem_out0, sem_out1,
):
    cid = lax.axis_index("c")
    sid = lax.axis_index("s")
    wid = sid * NUM_CORES + cid
    base0 = wid * TOK_PER_W

    idx = (idx0, idx1)
    rows = (rows0, rows1)
    sem_pre = (sem_pre0, sem_pre1)
    sem_idx = (sem_idx0, sem_idx1)
    sem_g = (sem_g0, sem_g1)
    sem_out = (sem_out0, sem_out1)

    @pl.when(sid == 0)
    def _():
        pltpu.sync_copy(pe_hbm, pe_sh)

    plsc.subcore_barrier()

    @pl.loop(0, SEQ_PER_W // 2)
    def _it(i):
        pres = []
        idxs = []
        for b in (0, 1):
            base = base0 + (2 * i + b) * SEQ

            @pl.when(i >= 1)
            def _(b=b, base=base):
                pltpu.make_async_copy(
                    rows[b], out_hbm.at[pl.ds(base, SEQ)], sem_out[b]
                ).wait()

            pres.append(pltpu.async_copy(pe_sh, rows[b], sem_pre[b]))
            idxs.append(
                pltpu.async_copy(x_hbm.at[pl.ds(base, SEQ)], idx[b], sem_idx[b])
            )

        gathers = []
        for b in (0, 1):
            pres[b].wait()
            idxs[b].wait()
            g0 = pltpu.async_copy(
                emb_hbm.at[idx[b].at[pl.ds(0, 128)]],
                rows[b].at[pl.ds(0, 128)],
                sem_g[b],
                add=True,
            )
            g1 = pltpu.async_copy(
                emb_hbm.at[idx[b].at[pl.ds(128, SEQ - 128)]],
                rows[b].at[pl.ds(128, SEQ - 128)],
                sem_g[b],
                add=True,
            )
            gathers.append((g0, g1))

        for b in (0, 1):
            g0, g1 = gathers[b]
            g0.wait()
            g1.wait()
            base = base0 + (2 * i + b) * SEQ
            pltpu.async_copy(rows[b], out_hbm.at[pl.ds(base, SEQ)], sem_out[b])

    for b in (0, 1):
        base = base0 + (SEQ_PER_W - 2 + b) * SEQ
        pltpu.make_async_copy(
            rows[b], out_hbm.at[pl.ds(base, SEQ)], sem_out[b]
        ).wait()


_PE = _positional_encoding()


def kernel(x, token_emb):
    out = _build_emb_kernel()(x.reshape(-1), token_emb, jnp.asarray(_PE))
    return out.reshape(BATCH, SEQ, D_MODEL)


# speedup vs baseline: 7.0571x; 1.0417x over previous
import math
import functools

import jax
import jax.numpy as jnp
import numpy as np
from jax import lax
from jax.experimental import pallas as pl
from jax.experimental.pallas import tpu as pltpu
from jax.experimental.pallas import tpu_sc as plsc

VOCAB = 100000
D_MODEL = 128
SEQ = 200
BATCH = 4096

NUM_CORES = 2
NUM_SUBCORES = 16
NUM_WORKERS = NUM_CORES * NUM_SUBCORES

TOKENS = BATCH * SEQ
TOK_PER_W = TOKENS // NUM_WORKERS
SEQ_PER_W = TOK_PER_W // SEQ
SEQS_PER_CHUNK = 2
CH = SEQS_PER_CHUNK * SEQ
CHUNKS_PER_W = TOK_PER_W // CH
GATHER_SPLITS = [(0, 128), (128, 128), (256, 128), (384, 16)]


def _positional_encoding():
    position = np.arange(0, SEQ, dtype=np.float64)[:, None]
    div_term = np.exp(
        np.arange(0, D_MODEL, 2, dtype=np.float64) * -(math.log(10000.0) / D_MODEL)
    )
    pe = np.zeros((SEQ, D_MODEL), dtype=np.float32)
    pe[:, 0::2] = np.sin(position * div_term).astype(np.float32)
    pe[:, 1::2] = np.cos(position * div_term).astype(np.float32)
    return pe


@functools.cache
def _build_emb_kernel():
    mesh = plsc.VectorSubcoreMesh(
        core_axis_name="c",
        subcore_axis_name="s",
        num_cores=NUM_CORES,
        num_subcores=NUM_SUBCORES,
    )
    return functools.partial(
        pl.kernel,
        out_type=jax.ShapeDtypeStruct((TOKENS, D_MODEL), jnp.float32),
        mesh=mesh,
        scratch_types=[
            pltpu.VMEM_SHARED((SEQ, D_MODEL), jnp.float32),
            pltpu.VMEM((CH,), jnp.int32),
            pltpu.VMEM((CH,), jnp.int32),
            pltpu.VMEM((CH, D_MODEL), jnp.float32),
            pltpu.VMEM((CH, D_MODEL), jnp.float32),
            pltpu.SemaphoreType.DMA,
            pltpu.SemaphoreType.DMA,
            pltpu.SemaphoreType.DMA,
            pltpu.SemaphoreType.DMA,
            pltpu.SemaphoreType.DMA,
            pltpu.SemaphoreType.DMA,
            pltpu.SemaphoreType.DMA,
            pltpu.SemaphoreType.DMA,
        ],
    )(_emb_body)


def _emb_body(
    x_hbm, emb_hbm, pe_hbm, out_hbm, pe_sh,
    idx0, idx1, rows0, rows1,
    sem_pre0, sem_pre1, sem_idx0, sem_idx1, sem_g0, sem_g1, sem_out0, sem_out1,
):
    cid = lax.axis_index("c")
    sid = lax.axis_index("s")
    wid = sid * NUM_CORES + cid
    base0 = wid * TOK_PER_W

    idx = (idx0, idx1)
    rows = (rows0, rows1)
    sem_pre = (sem_pre0, sem_pre1)
    sem_idx = (sem_idx0, sem_idx1)
    sem_g = (sem_g0, sem_g1)
    sem_out = (sem_out0, sem_out1)

    @pl.when(sid == 0)
    def _():
        pltpu.sync_copy(pe_hbm, pe_sh)

    plsc.subcore_barrier()

    @pl.loop(0, CHUNKS_PER_W // 2)
    def _it(i):
        pres = []
        idxs = []
        for b in (0, 1):
            base = base0 + (2 * i + b) * CH

            @pl.when(i >= 1)
            def _(b=b, base=base):
                pltpu.make_async_copy(
                    rows[b], out_hbm.at[pl.ds(base, CH)], sem_out[b]
                ).wait()

            pres.append([
                pltpu.async_copy(
                    pe_sh, rows[b].at[pl.ds(k * SEQ, SEQ)], sem_pre[b]
                )
                for k in range(SEQS_PER_CHUNK)
            ])
            idxs.append(
                pltpu.async_copy(x_hbm.at[pl.ds(base, CH)], idx[b], sem_idx[b])
            )

        gathers = []
        for b in (0, 1):
            for d in pres[b]:
                d.wait()
            idxs[b].wait()
            gathers.append([
                pltpu.async_copy(
                    emb_hbm.at[idx[b].at[pl.ds(off, n)]],
                    rows[b].at[pl.ds(off, n)],
                    sem_g[b],
                    add=True,
                )
                for off, n in GATHER_SPLITS
            ])

        for b in (0, 1):
            for d in gathers[b]:
                d.wait()
            base = base0 + (2 * i + b) * CH
            pltpu.async_copy(rows[b], out_hbm.at[pl.ds(base, CH)], sem_out[b])

    for b in (0, 1):
        base = base0 + (CHUNKS_PER_W - 2 + b) * CH
        pltpu.make_async_copy(
            rows[b], out_hbm.at[pl.ds(base, CH)], sem_out[b]
        ).wait()


_PE = _positional_encoding()


def kernel(x, token_emb):
    out = _build_emb_kernel()(x.reshape(-1), token_emb, jnp.asarray(_PE))
    return out.reshape(BATCH, SEQ, D_MODEL)


# speedup vs baseline: 8.0381x; 1.1390x over previous
import math
import functools

import jax
import jax.numpy as jnp
import numpy as np
from jax import lax
from jax.experimental import pallas as pl
from jax.experimental.pallas import tpu as pltpu
from jax.experimental.pallas import tpu_sc as plsc

VOCAB = 100000
D_MODEL = 128
SEQ = 200
BATCH = 4096

NUM_CORES = 2
NUM_SUBCORES = 16
NUM_WORKERS = NUM_CORES * NUM_SUBCORES

TOKENS = BATCH * SEQ
TOK_PER_W = TOKENS // NUM_WORKERS
SEQ_PER_W = TOK_PER_W // SEQ
SEQS_PER_CHUNK = 2
CH = SEQS_PER_CHUNK * SEQ
CHUNKS_PER_W = TOK_PER_W // CH
GATHER_SPLITS = [(0, 104), (104, 96), (200, 104), (304, 96)]


def _positional_encoding():
    position = np.arange(0, SEQ, dtype=np.float64)[:, None]
    div_term = np.exp(
        np.arange(0, D_MODEL, 2, dtype=np.float64) * -(math.log(10000.0) / D_MODEL)
    )
    pe = np.zeros((SEQ, D_MODEL), dtype=np.float32)
    pe[:, 0::2] = np.sin(position * div_term).astype(np.float32)
    pe[:, 1::2] = np.cos(position * div_term).astype(np.float32)
    return pe


@functools.cache
def _build_emb_kernel():
    mesh = plsc.VectorSubcoreMesh(
        core_axis_name="c",
        subcore_axis_name="s",
        num_cores=NUM_CORES,
        num_subcores=NUM_SUBCORES,
    )
    return functools.partial(
        pl.kernel,
        out_type=jax.ShapeDtypeStruct((TOKENS, D_MODEL), jnp.float32),
        mesh=mesh,
        scratch_types=[
            pltpu.VMEM_SHARED((SEQ, D_MODEL), jnp.float32),
            pltpu.VMEM((CH,), jnp.int32),
            pltpu.VMEM((CH,), jnp.int32),
            pltpu.VMEM((CH, D_MODEL), jnp.float32),
            pltpu.VMEM((CH, D_MODEL), jnp.float32),
            pltpu.SemaphoreType.DMA,
            pltpu.SemaphoreType.DMA,
            pltpu.SemaphoreType.DMA,
            pltpu.SemaphoreType.DMA,
            pltpu.SemaphoreType.DMA((4,)),
            pltpu.SemaphoreType.DMA((4,)),
            pltpu.SemaphoreType.DMA,
            pltpu.SemaphoreType.DMA,
        ],
    )(_emb_body)


def _emb_body(
    x_hbm, emb_hbm, pe_hbm, out_hbm, pe_sh,
    idx0, idx1, rows0, rows1,
    sem_pre0, sem_pre1, sem_idx0, sem_idx1, sem_g0, sem_g1, sem_out0, sem_out1,
):
    cid = lax.axis_index("c")
    sid = lax.axis_index("s")
    wid = sid * NUM_CORES + cid
    base0 = wid * TOK_PER_W

    idx = (idx0, idx1)
    rows = (rows0, rows1)
    sem_pre = (sem_pre0, sem_pre1)
    sem_idx = (sem_idx0, sem_idx1)
    sem_g = (sem_g0, sem_g1)
    sem_out = (sem_out0, sem_out1)

    @pl.when(sid == 0)
    def _():
        pltpu.sync_copy(pe_hbm, pe_sh)

    plsc.subcore_barrier()

    @pl.loop(0, CHUNKS_PER_W // 2)
    def _it(i):
        pres = []
        idxs = []
        for b in (0, 1):
            base = base0 + (2 * i + b) * CH

            @pl.when(i >= 1)
            def _(b=b, base=base):
                pltpu.make_async_copy(
                    rows[b], out_hbm.at[pl.ds(base, CH)], sem_out[b]
                ).wait()

            pres.append([
                pltpu.async_copy(
                    pe_sh, rows[b].at[pl.ds(k * SEQ, SEQ)], sem_pre[b]
                )
                for k in range(SEQS_PER_CHUNK)
            ])
            idxs.append(
                pltpu.async_copy(x_hbm.at[pl.ds(base, CH)], idx[b], sem_idx[b])
            )

        gathers = []
        for b in (0, 1):
            for d in pres[b]:
                d.wait()
            idxs[b].wait()
            gathers.append([
                pltpu.async_copy(
                    emb_hbm.at[idx[b].at[pl.ds(off, n)]],
                    rows[b].at[pl.ds(off, n)],
                    sem_g[b].at[k],
                    add=True,
                )
                for k, (off, n) in enumerate(GATHER_SPLITS)
            ])

        for b in (0, 1):
            base = base0 + (2 * i + b) * CH
            for k, (off, n) in enumerate(GATHER_SPLITS):
                gathers[b][k].wait()
                pltpu.async_copy(
                    rows[b].at[pl.ds(off, n)],
                    out_hbm.at[pl.ds(base + off, n)],
                    sem_out[b],
                )

    for b in (0, 1):
        base = base0 + (CHUNKS_PER_W - 2 + b) * CH
        pltpu.make_async_copy(
            rows[b], out_hbm.at[pl.ds(base, CH)], sem_out[b]
        ).wait()


_PE = _positional_encoding()


def kernel(x, token_emb):
    out = _build_emb_kernel()(x.reshape(-1), token_emb, jnp.asarray(_PE))
    return out.reshape(BATCH, SEQ, D_MODEL)


# speedup vs baseline: 8.9610x; 1.1148x over previous
import math
import functools

import jax
import jax.numpy as jnp
import numpy as np
from jax import lax
from jax.experimental import pallas as pl
from jax.experimental.pallas import tpu as pltpu
from jax.experimental.pallas import tpu_sc as plsc

VOCAB = 100000
D_MODEL = 128
SEQ = 200
BATCH = 4096

NUM_CORES = 2
NUM_SUBCORES = 16
NUM_WORKERS = NUM_CORES * NUM_SUBCORES

TOKENS = BATCH * SEQ
TOK_PER_W = TOKENS // NUM_WORKERS
SEQ_PER_W = TOK_PER_W // SEQ
SEQS_PER_CHUNK = 2
CH = SEQS_PER_CHUNK * SEQ
CHUNKS_PER_W = TOK_PER_W // CH
GATHER_SPLITS = [(0, 104), (104, 96), (200, 104), (304, 96)]


def _positional_encoding():
    position = np.arange(0, SEQ, dtype=np.float64)[:, None]
    div_term = np.exp(
        np.arange(0, D_MODEL, 2, dtype=np.float64) * -(math.log(10000.0) / D_MODEL)
    )
    pe = np.zeros((SEQ, D_MODEL), dtype=np.float32)
    pe[:, 0::2] = np.sin(position * div_term).astype(np.float32)
    pe[:, 1::2] = np.cos(position * div_term).astype(np.float32)
    return pe


@functools.cache
def _build_emb_kernel():
    mesh = plsc.VectorSubcoreMesh(
        core_axis_name="c",
        subcore_axis_name="s",
        num_cores=NUM_CORES,
        num_subcores=NUM_SUBCORES,
    )
    return functools.partial(
        pl.kernel,
        out_type=jax.ShapeDtypeStruct((TOKENS, D_MODEL), jnp.float32),
        mesh=mesh,
        scratch_types=[
            pltpu.VMEM_SHARED((SEQ, D_MODEL), jnp.float32),
            pltpu.VMEM((CH,), jnp.int32),
            pltpu.VMEM((CH,), jnp.int32),
            pltpu.VMEM((CH, D_MODEL), jnp.float32),
            pltpu.VMEM((CH, D_MODEL), jnp.float32),
            pltpu.SemaphoreType.DMA((4,)),
            pltpu.SemaphoreType.DMA((4,)),
            pltpu.SemaphoreType.DMA,
            pltpu.SemaphoreType.DMA,
            pltpu.SemaphoreType.DMA((4,)),
            pltpu.SemaphoreType.DMA((4,)),
            pltpu.SemaphoreType.DMA((4,)),
            pltpu.SemaphoreType.DMA((4,)),
        ],
    )(_emb_body)


def _emb_body(
    x_hbm, emb_hbm, pe_hbm, out_hbm, pe_sh,
    idx0, idx1, rows0, rows1,
    sem_pre0, sem_pre1, sem_idx0, sem_idx1, sem_g0, sem_g1, sem_out0, sem_out1,
):
    cid = lax.axis_index("c")
    sid = lax.axis_index("s")
    wid = sid * NUM_CORES + cid
    base0 = wid * TOK_PER_W

    idx = (idx0, idx1)
    rows = (rows0, rows1)
    sem_pre = (sem_pre0, sem_pre1)
    sem_idx = (sem_idx0, sem_idx1)
    sem_g = (sem_g0, sem_g1)
    sem_out = (sem_out0, sem_out1)

    @pl.when(sid == 0)
    def _():
        pltpu.sync_copy(pe_hbm, pe_sh)

    plsc.subcore_barrier()

    @pl.loop(0, CHUNKS_PER_W // 2)
    def _it(i):
        pres = []
        idxs = []
        for b in (0, 1):
            base = base0 + (2 * i + b) * CH
            for k, (off, n) in enumerate(GATHER_SPLITS):
                @pl.when(i >= 1)
                def _(b=b, k=k, off=off, n=n, base=base):
                    pltpu.make_async_copy(
                        rows[b].at[pl.ds(off, n)],
                        out_hbm.at[pl.ds(base + off, n)],
                        sem_out[b].at[k],
                    ).wait()

            pres.append([
                pltpu.async_copy(
                    pe_sh.at[pl.ds(off % SEQ, n)],
                    rows[b].at[pl.ds(off, n)],
                    sem_pre[b].at[k],
                )
                for k, (off, n) in enumerate(GATHER_SPLITS)
            ])
            idxs.append(
                pltpu.async_copy(x_hbm.at[pl.ds(base, CH)], idx[b], sem_idx[b])
            )

        gathers = []
        for b in (0, 1):
            idxs[b].wait()
            gs = []
            for k, (off, n) in enumerate(GATHER_SPLITS):
                pres[b][k].wait()
                gs.append(
                    pltpu.async_copy(
                        emb_hbm.at[idx[b].at[pl.ds(off, n)]],
                        rows[b].at[pl.ds(off, n)],
                        sem_g[b].at[k],
                        add=True,
                    )
                )
            gathers.append(gs)

        for b in (0, 1):
            base = base0 + (2 * i + b) * CH
            for k, (off, n) in enumerate(GATHER_SPLITS):
                gathers[b][k].wait()
                pltpu.async_copy(
                    rows[b].at[pl.ds(off, n)],
                    out_hbm.at[pl.ds(base + off, n)],
                    sem_out[b].at[k],
                )

    for b in (0, 1):
        base = base0 + (CHUNKS_PER_W - 2 + b) * CH
        for k, (off, n) in enumerate(GATHER_SPLITS):
            pltpu.make_async_copy(
                rows[b].at[pl.ds(off, n)],
                out_hbm.at[pl.ds(base + off, n)],
                sem_out[b].at[k],
            ).wait()


_PE = _positional_encoding()


def kernel(x, token_emb):
    out = _build_emb_kernel()(x.reshape(-1), token_emb, jnp.asarray(_PE))
    return out.reshape(BATCH, SEQ, D_MODEL)
